# Initial kernel scaffold; baseline (speedup 1.0000x reference)
import jax
import jax.numpy as jnp
from jax.experimental import pallas as pl


def kernel(loc_preds, cls_preds, loc_targets, cls_targets):
    raise NotImplementedError("write your pallas kernel here")


# speedup vs baseline: 24.2012x; 24.2012x over previous
import functools

import jax
import jax.numpy as jnp
from jax import lax
from jax.experimental import pallas as pl
from jax.experimental.pallas import tpu as pltpu
from jax.experimental.pallas import tpu_sc as plsc

_INF_BITS = 0x7F800001


def _phase1_body(cp_ref, t0r_ref, m4_ref, lp_ref, lt_ref, ncl_ref, st_ref):
    x = cp_ref[0].astype(jnp.float32)
    d = x - x[0:1, :]
    e = jnp.exp(d)
    s = jnp.sum(e, axis=0, keepdims=True)
    ncl = jnp.log(s)
    posr = t0r_ref[0] == 1
    ncl_ref[0] = jnp.where(posr, -0.0, ncl)

    m4 = m4_ref[0].astype(jnp.float32)
    dd = lp_ref[0] - lt_ref[0]
    ad = jnp.abs(dd)
    sl1 = jnp.where(ad < 1.0, 0.5 * dd * dd, ad - 0.5)
    locp = jnp.sum(sl1 * m4)

    lane = jax.lax.broadcasted_iota(jnp.int32, (1, 1, 128), 2)
    st_ref[...] = jnp.where(lane == 0, locp, 0.0)


def _phase2_body(p_total, ncl_ref, st_ref, out_ref):
    v = ncl_ref[...]
    bsz = v.shape[0]
    loc_loss = jnp.sum(st_ref[:, 0:1])
    vbits = jax.lax.bitcast_convert_type(v, jnp.int32)
    npos = jnp.sum((vbits < 0).astype(jnp.int32), axis=1, keepdims=True)
    ki = 3 * npos
    kf = ki.astype(jnp.float32)
    s_total = jnp.sum(v, axis=1, keepdims=True)

    def search(_):
        lo = jnp.zeros((bsz, 1), jnp.int32)
        hi = jnp.full((bsz, 1), _INF_BITS, jnp.int32)

        def body(_, carry):
            lo, hi = carry
            mid = lo + (hi - lo) // 2
            t = jax.lax.bitcast_convert_type(mid, jnp.float32)
            cnt_ge = jnp.sum((v >= t).astype(jnp.int32), axis=1, keepdims=True)
            ok = cnt_ge >= ki
            return jnp.where(ok, mid, lo), jnp.where(ok, hi, mid)

        lo, hi = jax.lax.fori_loop(0, 31, body, (lo, hi))
        t = jax.lax.bitcast_convert_type(lo, jnp.float32)
        gt = v > t
        cnt_gt = jnp.sum(gt.astype(jnp.float32), axis=1, keepdims=True)
        s_gt = jnp.sum(jnp.where(gt, v, 0.0), axis=1, keepdims=True)
        return s_gt + t * (kf - cnt_gt)

    need_search = jnp.any((ki > 0) & (ki < p_total))
    s_sel = jax.lax.cond(need_search, search, lambda _: s_total, None)
    s_row = jnp.where(ki <= 0, 0.0, jnp.where(ki >= p_total, s_total, s_sel))
    out_ref[...] = jnp.zeros((8, 128), jnp.float32) + (jnp.sum(s_row) + loc_loss)


def _make_sc_topk(bsz, p):
    nw = 32
    rows_per_w = bsz // nw
    nv = p // 16
    inner = 10
    outer = nv // inner

    @functools.partial(
        pl.kernel,
        mesh=plsc.VectorSubcoreMesh(core_axis_name="c", subcore_axis_name="s"),
        out_type=jax.ShapeDtypeStruct((bsz, 16), jnp.float32),
        scratch_types=[pltpu.VMEM((p,), jnp.float32),
                       pltpu.VMEM((16,), jnp.float32)],
    )
    def sc_topk(ncl_hbm, out_hbm, row_v, out_v):
        wid = lax.axis_index("s") * 2 + lax.axis_index("c")
        for r in range(rows_per_w):
            row = wid * rows_per_w + r
            pltpu.sync_copy(ncl_hbm.at[row], row_v)

            def pass_a(i, carry):
                tot, npos = carry
                for j in range(inner):
                    v = row_v[pl.ds((i * inner + j) * 16, 16)]
                    bits = lax.bitcast_convert_type(v, jnp.int32)
                    tot = tot + v
                    npos = npos + jnp.where(
                        bits < 0, jnp.ones((16,), jnp.int32),
                        jnp.zeros((16,), jnp.int32))
                return tot, npos

            tot_v, npos_v = lax.fori_loop(
                0, outer, pass_a,
                (jnp.zeros((16,), jnp.float32), jnp.zeros((16,), jnp.int32)))
            npos_s = npos_v[0]
            for j in range(1, 16):
                npos_s = npos_s + npos_v[j]
            ki = 3 * npos_s
            kf = ki.astype(jnp.float32)

            def search():
                def step(_, carry):
                    lo, hi = carry
                    mid = lo + lax.shift_right_arithmetic(
                        hi - lo, jnp.ones((16,), jnp.int32))
                    t = lax.bitcast_convert_type(mid, jnp.float32)

                    def body(i, acc):
                        for j in range(inner):
                            v = row_v[pl.ds((i * inner + j) * 16, 16)]
                            acc = acc + jnp.where(
                                v >= t, jnp.ones((16,), jnp.int32),
                                jnp.zeros((16,), jnp.int32))
                        return acc

                    cnt_v = lax.fori_loop(0, outer, body,
                                          jnp.zeros((16,), jnp.int32))
                    cnt_s = cnt_v[0]
                    for j in range(1, 16):
                        cnt_s = cnt_s + cnt_v[j]
                    okv = lax.broadcast_in_dim(
                        (cnt_s >= ki).astype(jnp.int32), (16,), ())
                    return lo + (mid - lo) * okv, mid + (hi - mid) * okv

                lo, _ = lax.fori_loop(
                    0, 31, step,
                    (jnp.zeros((16,), jnp.int32),
                     jnp.full((16,), _INF_BITS, jnp.int32)))
                t = lax.bitcast_convert_type(lo, jnp.float32)

                def tail(i, carry):
                    sgt, cgt = carry
                    for j in range(inner):
                        v = row_v[pl.ds((i * inner + j) * 16, 16)]
                        gt = v > t
                        sgt = sgt + jnp.where(gt, v,
                                               jnp.zeros((16,), jnp.float32))
                        cgt = cgt + jnp.where(
                            gt, jnp.ones((16,), jnp.int32),
                            jnp.zeros((16,), jnp.int32))
                    return sgt, cgt

                sgt_v, cgt_v = lax.fori_loop(
                    0, outer, tail,
                    (jnp.zeros((16,), jnp.float32), jnp.zeros((16,), jnp.int32)))
                cg_s = cgt_v[0]
                for j in range(1, 16):
                    cg_s = cg_s + cgt_v[j]
                corr = (kf - cg_s.astype(jnp.float32)) * (1.0 / 16.0)
                return sgt_v + t * lax.broadcast_in_dim(corr, (16,), ())

            need = (ki > 0) & (ki < p)
            gate = jnp.where(ki > 0, 1.0, 0.0).astype(jnp.float32)

            @pl.when(jnp.logical_not(need))
            def _():
                out_v[...] = tot_v * lax.broadcast_in_dim(gate, (16,), ())

            @pl.when(need)
            def _():
                out_v[...] = search()

            pltpu.sync_copy(out_v, out_hbm.at[row])

    return sc_topk


def kernel(loc_preds, cls_preds, loc_targets, cls_targets):
    bsz, p, c = cls_preds.shape
    ga = p // 32

    cp_t = jnp.transpose(cls_preds.astype(jnp.bfloat16), (0, 2, 1))
    notbg = (cls_targets[:, :, 0] != 1).astype(jnp.int8)
    tgt0_row = notbg.reshape(bsz, 1, p)
    mask4 = jnp.repeat(notbg, 4, axis=1).reshape(bsz, ga, 128)
    lp = loc_preds.reshape(bsz, ga, 128)
    lt = loc_targets.reshape(bsz, ga, 128)

    ncl, stats = pl.pallas_call(
        _phase1_body,
        grid=(bsz,),
        in_specs=[
            pl.BlockSpec((1, c, p), lambda b: (b, 0, 0)),
            pl.BlockSpec((1, 1, p), lambda b: (b, 0, 0)),
            pl.BlockSpec((1, ga, 128), lambda b: (b, 0, 0)),
            pl.BlockSpec((1, ga, 128), lambda b: (b, 0, 0)),
            pl.BlockSpec((1, ga, 128), lambda b: (b, 0, 0)),
        ],
        out_specs=[
            pl.BlockSpec((1, 1, p), lambda b: (b, 0, 0)),
            pl.BlockSpec((1, 1, 128), lambda b: (b, 0, 0)),
        ],
        out_shape=[
            jax.ShapeDtypeStruct((bsz, 1, p), jnp.float32),
            jax.ShapeDtypeStruct((bsz, 1, 128), jnp.float32),
        ],
    )(cp_t, tgt0_row, mask4, lp, lt)

    sc_out = _make_sc_topk(bsz, p)(ncl.reshape(bsz, p))
    return jnp.sum(sc_out) + jnp.sum(stats.reshape(bsz, 128)[:, 0])
